# baseline (speedup 1.0000x reference)
import functools
import math

import jax
import jax.numpy as jnp
from jax import lax
from jax.experimental import pallas as pl
from jax.experimental.pallas import tpu as pltpu
from jax.experimental.pallas import tpu_sc as plsc

N = 10000
NPAD = 10240
E = 160000
EPAD = 163840
G = 128
F = 32
NTILES = 32
EDGES_PER_TILE = EPAD // NTILES
NCHUNK = 40
CB = 128
ROWS_PER_TILE = NPAD // NTILES
KS = [5000, 4000, 3200, 2560]
NEG_INF = float("-inf")
INT_MIN = -2147483648


def _bdot(a, b):
    return jnp.dot(a.astype(jnp.bfloat16), b.astype(jnp.bfloat16),
                   preferred_element_type=jnp.float32)


def _gmlp_body(e_ref, w0, w1, w2, w3, w4, b0, b1, b2, b3, b4,
               s0, s1, s2, s3, B0, B1, B2, B3, o_ref):
    h = e_ref[...]
    wsr = (w0, w1, w2, w3)
    bsr = (b0, b1, b2, b3)
    ssr = (s0, s1, s2, s3)
    Bsr = (B0, B1, B2, B3)
    for l in range(4):
        h = _bdot(h, wsr[l][...]) + bsr[l][...]
        h = h * ssr[l][...] + Bsr[l][...]
        h = jnp.maximum(h, 0.0)
    o_ref[...] = _bdot(h, w4[...]) + b4[...]


def _gmlp(e_pad, ws, bs, ss, Bs):
    blk = 1024
    grid = EPAD // blk
    full = lambda shape: pl.BlockSpec(shape, lambda i: (0, 0))
    return pl.pallas_call(
        _gmlp_body,
        grid=(grid,),
        in_specs=[pl.BlockSpec((blk, F), lambda i: (i, 0))]
        + [full(w.shape) for w in ws] + [full(b.shape) for b in bs]
        + [full(s.shape) for s in ss] + [full(B.shape) for B in Bs],
        out_specs=pl.BlockSpec((blk, F), lambda i: (i, 0)),
        out_shape=jax.ShapeDtypeStruct((EPAD, F), jnp.float32),
        compiler_params=pltpu.CompilerParams(
            dimension_semantics=("arbitrary",)),
    )(e_pad, *ws, *bs, *ss, *Bs)


NBUF = 4


def _scan_rows(wid, xv, selv, bv, pm):
    ninf = jnp.full((16,), NEG_INF, jnp.float32)

    def _init(i, _):
        pm[pl.ds(i * 16, 16)] = ninf
        return 0
    lax.fori_loop(0, G * F // 16, _init, 0)

    def _group(gi, carry):
        cur_g, m0, m1 = carry
        vb = bv[pl.ds(gi * 16, 16)]
        vs = selv[pl.ds(gi * 16, 16)]
        for j in range(16):
            r = gi * 16 + j
            g = vb[j]
            svaln = vs[j]
            x0 = xv[r, 0:16]
            x1 = xv[r, 16:32]
            x0 = jnp.where(svaln > 0, x0, ninf)
            x1 = jnp.where(svaln > 0, x1, ninf)
            is_new = g != cur_g

            @pl.when(is_new & (cur_g >= 0))
            def _():
                pm[pl.ds(cur_g * F, 16)] = m0
                pm[pl.ds(cur_g * F + 16, 16)] = m1

            m0 = jnp.where(is_new, x0, jnp.maximum(m0, x0))
            m1 = jnp.where(is_new, x1, jnp.maximum(m1, x1))
            cur_g = g
        return (cur_g, m0, m1)

    cur_g, m0, m1 = lax.fori_loop(
        0, ROWS_PER_TILE // 16, _group, (jnp.int32(-1), ninf, ninf))

    @pl.when(cur_g >= 0)
    def _():
        pm[pl.ds(cur_g * F, 16)] = m0
        pm[pl.ds(cur_g * F + 16, 16)] = m1


def _conv_body(do_scan, *refs):
    if do_scan:
        (x_hbm, w_hbm, src_hbm, dst_hbm, sel_hbm, bat_hbm,
         out_hbm, pmout_hbm) = refs[:8]
        refs = refs[8:]
        xv, selv, bv, pm = refs[:4]
        refs = refs[4:]
    else:
        x_hbm, w_hbm, src_hbm, dst_hbm, out_hbm = refs[:5]
        refs = refs[5:]
    src_v, dst_v, zb, acc = refs[:4]
    refs = refs[4:]
    xbufs = refs[0:NBUF]
    wbufs = refs[NBUF:2 * NBUF]
    gsems = refs[2 * NBUF:3 * NBUF]
    wsems = refs[3 * NBUF:4 * NBUF]
    ssems = refs[4 * NBUF:5 * NBUF]

    c = lax.axis_index("c")
    s = lax.axis_index("s")
    wid = s * 2 + c
    base = wid * EDGES_PER_TILE

    pltpu.sync_copy(src_hbm.at[wid], src_v)
    pltpu.sync_copy(dst_hbm.at[wid], dst_v)

    def _gcp(chunk, b):
        return pltpu.make_async_copy(x_hbm.at[src_v.at[chunk]], xbufs[b], gsems[b])

    def _wcp(chunk, b):
        return pltpu.make_async_copy(
            w_hbm.at[pl.ds(base + chunk * CB, CB)], wbufs[b], wsems[b])

    def _scp_start(chunk, b):
        pltpu.async_copy(xbufs[b], acc.at[dst_v.at[chunk]], ssems[b], add=True)

    def _scp_wait(chunk, b):
        pltpu.make_async_copy(
            xbufs[b], acc.at[dst_v.at[chunk]], ssems[b]).wait()

    _gcp(0, 0).start()
    _wcp(0, 0).start()

    if do_scan:
        nbase = wid * ROWS_PER_TILE
        pltpu.sync_copy(x_hbm.at[pl.ds(nbase, ROWS_PER_TILE)], xv)
        pltpu.sync_copy(sel_hbm.at[pl.ds(nbase, ROWS_PER_TILE)], selv)
        pltpu.sync_copy(bat_hbm.at[pl.ds(nbase, ROWS_PER_TILE)], bv)

    def _z(i, _):
        zb[i, 0:16] = jnp.zeros((16,), jnp.float32)
        zb[i, 16:32] = jnp.zeros((16,), jnp.float32)
        return 0
    lax.fori_loop(0, CB, _z, 0)
    for q in range(ROWS_PER_TILE * 2 // CB):
        pltpu.sync_copy(zb, acc.at[pl.ds(s * 640 + q * CB, CB)])
    plsc.subcore_barrier()

    if do_scan:
        _scan_rows(wid, xv, selv, bv, pm)
        pltpu.sync_copy(pm, pmout_hbm.at[wid])

    def _mul(b):
        xb, wb = xbufs[b], wbufs[b]

        def _m(r, _):
            xb[r, 0:16] = xb[r, 0:16] * wb[r, 0:16]
            xb[r, 16:32] = xb[r, 16:32] * wb[r, 16:32]
            return 0
        lax.fori_loop(0, CB, _m, 0, unroll=8)

    def _outer(jj, _):
        for b in range(NBUF):
            chunk = jj * NBUF + b
            nxt = chunk + 1
            nb = (b + 1) % NBUF

            @pl.when(chunk >= NBUF - 1)
            def _():
                _scp_wait(chunk - (NBUF - 1), nb)

            @pl.when(nxt < NCHUNK)
            def _():
                _gcp(nxt, nb).start()
                _wcp(nxt, nb).start()
            _gcp(chunk, b).wait()
            _wcp(chunk, b).wait()
            _mul(b)
            _scp_start(chunk, b)
        return 0

    lax.fori_loop(0, NCHUNK // NBUF, _outer, 0)
    for tail in range(NCHUNK - (NBUF - 1), NCHUNK):
        _scp_wait(tail, tail % NBUF)

    plsc.subcore_barrier()
    pltpu.sync_copy(acc.at[pl.ds(s * 640, 640)],
                    out_hbm.at[c, pl.ds(s * 640, 640)])


def _conv_scratch():
    return ([
        pltpu.VMEM((NCHUNK, CB), jnp.int32),
        pltpu.VMEM((NCHUNK, CB), jnp.int32),
        pltpu.VMEM((CB, F), jnp.float32),
        pltpu.VMEM_SHARED((NPAD, F), jnp.float32),
    ] + [pltpu.VMEM((CB, F), jnp.float32)] * (2 * NBUF)
      + [pltpu.SemaphoreType.DMA] * (3 * NBUF))


def _conv_sc(x_pad, w, src_t, dst_t):
    mesh = plsc.VectorSubcoreMesh(core_axis_name="c", subcore_axis_name="s")
    return pl.kernel(
        functools.partial(_conv_body, False),
        out_type=jax.ShapeDtypeStruct((2, NPAD, F), jnp.float32),
        mesh=mesh,
        compiler_params=pltpu.CompilerParams(use_tc_tiling_on_sc=False),
        scratch_types=_conv_scratch(),
    )(x_pad, w, src_t, dst_t)


def _conv_scan_sc(x_pad, sel1d, bat1d, w, src_t, dst_t):
    mesh = plsc.VectorSubcoreMesh(core_axis_name="c", subcore_axis_name="s")
    return pl.kernel(
        functools.partial(_conv_body, True),
        out_type=[
            jax.ShapeDtypeStruct((2, NPAD, F), jnp.float32),
            jax.ShapeDtypeStruct((NTILES, G * F), jnp.float32),
        ],
        mesh=mesh,
        compiler_params=pltpu.CompilerParams(use_tc_tiling_on_sc=False),
        scratch_types=[
            pltpu.VMEM((ROWS_PER_TILE, F), jnp.float32),
            pltpu.VMEM((ROWS_PER_TILE,), jnp.int32),
            pltpu.VMEM((ROWS_PER_TILE,), jnp.int32),
            pltpu.VMEM((G * F,), jnp.float32),
        ] + _conv_scratch(),
    )(x_pad, w, src_t, dst_t, sel1d, bat1d)


def _node_body(kk, part, batT, alive, sref, bref, pwref,
               xnext_ref, sel_ref, mean_ref):
    agg = part[0] + part[1]
    hb = jnp.maximum(agg, 0.0) * sref[...] + bref[...]
    pw = pwref[...]
    norm = jnp.sqrt(jnp.sum(pw * pw))
    sdot = lax.dot_general(pw.astype(jnp.bfloat16), hb.astype(jnp.bfloat16),
                           (((1,), (1,)), ((), ())),
                           preferred_element_type=jnp.float32)
    score = sdot / norm

    bits = lax.bitcast_convert_type(score, jnp.int32)
    key = jnp.where(bits < 0,
                    jnp.bitwise_xor(jnp.bitwise_not(bits), jnp.int32(INT_MIN)),
                    bits)
    key = jnp.where(alive[...] > 0, key, jnp.int32(INT_MIN))

    prefix = jnp.int32(INT_MIN)
    for b in range(31, -1, -1):
        if b == 31:
            cand = jnp.bitwise_xor(prefix, jnp.int32(INT_MIN))
        else:
            cand = jnp.bitwise_or(prefix, jnp.int32(1 << b))
        c = jnp.sum((key >= cand).astype(jnp.int32))
        prefix = jnp.where(c >= kk, cand, prefix)
    t = prefix

    gt = key > t
    eq = key == t
    need = jnp.int32(kk) - jnp.sum(gt.astype(jnp.int32))
    idx = lax.broadcasted_iota(jnp.int32, (1, NPAD), 1)
    pref = jnp.int32(0)
    for b in range(13, -1, -1):
        cand = jnp.bitwise_or(pref, jnp.int32(1 << b))
        c = jnp.sum((eq & (idx < cand)).astype(jnp.int32))
        pref = jnp.where(c < need, cand, pref)
    sel = gt | (eq & (idx <= pref) & (need > 0))

    mult = jnp.where(sel, jnp.tanh(score), 0.0)
    ones11 = jnp.ones((1, 1), jnp.float32)
    hi = jax.lax.Precision.HIGHEST
    multT = lax.dot_general(mult, ones11, (((0,), (0,)), ((), ())),
                            precision=hi,
                            preferred_element_type=jnp.float32)
    xnext = hb * multT
    xnext_ref[...] = xnext
    sel_ref[...] = sel.astype(jnp.int32)

    sel01 = sel.astype(jnp.float32)
    selT = lax.dot_general(sel01, ones11, (((0,), (0,)), ((), ())),
                           precision=hi,
                           preferred_element_type=jnp.float32)
    giota = lax.broadcasted_iota(jnp.int32, (NPAD, G), 1)
    onehot = (batT[...] == giota).astype(jnp.float32)
    sm = lax.dot_general(onehot, xnext, (((0,), (0,)), ((), ())),
                         precision=hi,
                         preferred_element_type=jnp.float32)
    cnt = lax.dot_general(onehot, selT, (((0,), (0,)), ((), ())),
                          precision=hi,
                          preferred_element_type=jnp.float32)
    mean_ref[...] = sm / jnp.maximum(cnt, 1.0)


def _node_tc(kk, part, batT, alive, s_i, b_i, pw_i):
    return pl.pallas_call(
        functools.partial(_node_body, kk),
        out_shape=[
            jax.ShapeDtypeStruct((NPAD, F), jnp.float32),
            jax.ShapeDtypeStruct((1, NPAD), jnp.int32),
            jax.ShapeDtypeStruct((G, F), jnp.float32),
        ],
    )(part, batT, alive, s_i, b_i, pw_i)


def _gmax_body(x_hbm, sel_hbm, bat_hbm, out_hbm, xv, selv, bv, pm):
    c = lax.axis_index("c")
    s = lax.axis_index("s")
    wid = s * 2 + c
    base = wid * ROWS_PER_TILE

    pltpu.sync_copy(x_hbm.at[pl.ds(base, ROWS_PER_TILE)], xv)
    pltpu.sync_copy(sel_hbm.at[pl.ds(base, ROWS_PER_TILE)], selv)
    pltpu.sync_copy(bat_hbm.at[pl.ds(base, ROWS_PER_TILE)], bv)

    _scan_rows(wid, xv, selv, bv, pm)

    pltpu.sync_copy(pm, out_hbm.at[wid])


def _gmax_sc(x_pad, sel1d, bat1d):
    mesh = plsc.VectorSubcoreMesh(core_axis_name="c", subcore_axis_name="s")
    return pl.kernel(
        _gmax_body,
        out_type=jax.ShapeDtypeStruct((NTILES, G * F), jnp.float32),
        mesh=mesh,
        compiler_params=pltpu.CompilerParams(use_tc_tiling_on_sc=False),
        scratch_types=[
            pltpu.VMEM((ROWS_PER_TILE, F), jnp.float32),
            pltpu.VMEM((ROWS_PER_TILE,), jnp.int32),
            pltpu.VMEM((ROWS_PER_TILE,), jnp.int32),
            pltpu.VMEM((G * F,), jnp.float32),
        ],
    )(x_pad, sel1d, bat1d)


def _head_body(pm0, pm1, pm2, pm3, mn0, mn1, mn2, mn3,
               w1, b1, s1, B1, w2, b2, s2, B2, w3, b3, o_ref):
    pieces = []
    for pm, mn in ((pm0, mn0), (pm1, mn1), (pm2, mn2), (pm3, mn3)):
        mx = jnp.max(pm[...], axis=0)
        mx = jnp.where(mx > jnp.float32(NEG_INF), mx, 0.0)
        pieces.append(mx)
        pieces.append(mn[...])
    z = jnp.concatenate(pieces, axis=1)
    h = jnp.maximum(_bdot(z, w1[...]) + b1[...], 0.0)
    h = h * s1[...] + B1[...]
    h = jnp.maximum(_bdot(h, w2[...]) + b2[...], 0.0)
    h = h * s2[...] + B2[...]
    o_ref[...] = _bdot(h, w3[...]) + b3[...]


def _head_tc(pms, mns, hw):
    return pl.pallas_call(
        _head_body,
        out_shape=jax.ShapeDtypeStruct((G, 1), jnp.float32),
    )(*pms, *mns, *hw)


def kernel(x, edge_index, batch, weight, params):
    p = params
    f32 = jnp.float32
    sqc = jnp.sqrt(jnp.float32(1.0 + 1e-5))

    ws, bs, ss, Bs = [], [], [], []
    for l in range(5):
        ws.append(p["gW%d" % l].T.astype(f32))
        bs.append(p["gb%d" % l].astype(f32).reshape(1, -1))
        if l < 4:
            ss.append((p["gg%d" % l] / sqc).reshape(1, -1))
            Bs.append(p["gB%d" % l].reshape(1, -1))

    e_pad = jnp.concatenate(
        [weight, jnp.zeros((EPAD - E, F), f32)], axis=0)
    w = _gmlp(e_pad, ws, bs, ss, Bs)

    src = edge_index[0].astype(jnp.int32)
    dst = edge_index[1].astype(jnp.int32)
    pad_idx = jnp.full((EPAD - E,), N, jnp.int32)
    src_t = jnp.concatenate([src, pad_idx]).reshape(NTILES, NCHUNK, CB)
    dst_t = jnp.concatenate([dst, pad_idx]).reshape(NTILES, NCHUNK, CB)

    x_pad = jnp.zeros((NPAD, F), f32).at[:N].set(x.astype(f32))
    bat1d = jnp.zeros((NPAD,), jnp.int32).at[:N].set(batch.astype(jnp.int32))
    batT = bat1d[:, None]
    alive = (jnp.arange(NPAD, dtype=jnp.int32)[None, :] < N).astype(jnp.int32)

    pms, mns = [], []
    xs = x_pad
    sel = None
    for i in range(4):
        if i == 0:
            part = _conv_sc(xs, w, src_t, dst_t)
        else:
            part, pm_prev = _conv_scan_sc(
                xs, sel.reshape(NPAD), bat1d, w, src_t, dst_t)
            pms.append(pm_prev.reshape(NTILES, G, F))
        s_i = (p["bng%d" % (i + 1)] / sqc).reshape(1, F)
        b_i = p["bnb%d" % (i + 1)].reshape(1, F)
        pw_i = p["pw%d" % (i + 1)].reshape(1, F).astype(f32)
        xs, sel, mean_i = _node_tc(KS[i], part, batT, alive, s_i, b_i, pw_i)
        alive = sel
        mns.append(mean_i)
    pms.append(_gmax_sc(xs, sel.reshape(NPAD), bat1d).reshape(NTILES, G, F))

    hw = [
        p["lin1W"].T.astype(f32), p["lin1b"].reshape(1, -1),
        (p["bn1g"] / sqc).reshape(1, -1), p["bn1b"].reshape(1, -1),
        p["lin2W"].T.astype(f32), p["lin2b"].reshape(1, -1),
        (p["bn2g"] / sqc).reshape(1, -1), p["bn2b"].reshape(1, -1),
        p["lin3W"].T.astype(f32), p["lin3b"].reshape(1, -1),
    ]
    out = _head_tc(pms, mns, hw)
    return out.reshape(-1)


# speedup vs baseline: 1.0940x; 1.0940x over previous
import functools
import math

import jax
import jax.numpy as jnp
from jax import lax
from jax.experimental import pallas as pl
from jax.experimental.pallas import tpu as pltpu
from jax.experimental.pallas import tpu_sc as plsc

N = 10000
NPAD = 10240
E = 160000
EPAD = 163840
G = 128
F = 32
NTILES = 32
EDGES_PER_TILE = EPAD // NTILES
NCHUNK = 40
CB = 128
ROWS_PER_TILE = NPAD // NTILES
KS = [5000, 4000, 3200, 2560]
NEG_INF = float("-inf")
INT_MIN = -2147483648


def _bdot(a, b):
    return jnp.dot(a.astype(jnp.bfloat16), b.astype(jnp.bfloat16),
                   preferred_element_type=jnp.float32)


def _gmlp_body(e_ref, w0, w1, w2, w3, w4, b0, b1, b2, b3, b4,
               s0, s1, s2, s3, B0, B1, B2, B3, o_ref):
    h = e_ref[...]
    wsr = (w0, w1, w2, w3)
    bsr = (b0, b1, b2, b3)
    ssr = (s0, s1, s2, s3)
    Bsr = (B0, B1, B2, B3)
    for l in range(4):
        h = _bdot(h, wsr[l][...]) + bsr[l][...]
        h = h * ssr[l][...] + Bsr[l][...]
        h = jnp.maximum(h, 0.0)
    o_ref[...] = _bdot(h, w4[...]) + b4[...]


def _gmlp(e_pad, ws, bs, ss, Bs):
    blk = 1024
    grid = EPAD // blk
    full = lambda shape: pl.BlockSpec(shape, lambda i: (0, 0))
    return pl.pallas_call(
        _gmlp_body,
        grid=(grid,),
        in_specs=[pl.BlockSpec((blk, F), lambda i: (i, 0))]
        + [full(w.shape) for w in ws] + [full(b.shape) for b in bs]
        + [full(s.shape) for s in ss] + [full(B.shape) for B in Bs],
        out_specs=pl.BlockSpec((blk, F), lambda i: (i, 0)),
        out_shape=jax.ShapeDtypeStruct((EPAD, F), jnp.float32),
        compiler_params=pltpu.CompilerParams(
            dimension_semantics=("arbitrary",)),
    )(e_pad, *ws, *bs, *ss, *Bs)


NBUF = 4


def _scan_rows(wid, xv, selv, bv, pm):
    ninf = jnp.full((16,), NEG_INF, jnp.float32)

    def _init(i, _):
        pm[pl.ds(i * 16, 16)] = ninf
        return 0
    lax.fori_loop(0, G * F // 16, _init, 0)

    def _group(gi, carry):
        cur_g, m0, m1 = carry
        vb = bv[pl.ds(gi * 16, 16)]
        vs = selv[pl.ds(gi * 16, 16)]
        for j in range(16):
            r = gi * 16 + j
            g = vb[j]
            svaln = vs[j]
            x0 = xv[r, 0:16]
            x1 = xv[r, 16:32]
            x0 = jnp.where(svaln > 0, x0, ninf)
            x1 = jnp.where(svaln > 0, x1, ninf)
            is_new = g != cur_g

            @pl.when(is_new & (cur_g >= 0))
            def _():
                pm[pl.ds(cur_g * F, 16)] = m0
                pm[pl.ds(cur_g * F + 16, 16)] = m1

            m0 = jnp.where(is_new, x0, jnp.maximum(m0, x0))
            m1 = jnp.where(is_new, x1, jnp.maximum(m1, x1))
            cur_g = g
        return (cur_g, m0, m1)

    cur_g, m0, m1 = lax.fori_loop(
        0, ROWS_PER_TILE // 16, _group, (jnp.int32(-1), ninf, ninf))

    @pl.when(cur_g >= 0)
    def _():
        pm[pl.ds(cur_g * F, 16)] = m0
        pm[pl.ds(cur_g * F + 16, 16)] = m1


def _conv_body(do_scan, *refs):
    if do_scan:
        (x_hbm, w_hbm, src_hbm, dst_hbm, sel_hbm, bat_hbm,
         out_hbm, pmout_hbm) = refs[:8]
        refs = refs[8:]
        xv, selv, bv, pm = refs[:4]
        refs = refs[4:]
    else:
        x_hbm, w_hbm, src_hbm, dst_hbm, out_hbm = refs[:5]
        refs = refs[5:]
    src_v, dst_v, zb, acc, x_sh = refs[:5]
    refs = refs[5:]
    xbufs = refs[0:NBUF]
    wbufs = refs[NBUF:2 * NBUF]
    gsems = refs[2 * NBUF:3 * NBUF]
    wsems = refs[3 * NBUF:4 * NBUF]
    ssems = refs[4 * NBUF:5 * NBUF]

    c = lax.axis_index("c")
    s = lax.axis_index("s")
    wid = s * 2 + c
    base = wid * EDGES_PER_TILE

    pltpu.sync_copy(src_hbm.at[wid], src_v)
    pltpu.sync_copy(dst_hbm.at[wid], dst_v)
    pltpu.sync_copy(x_hbm.at[pl.ds(s * 640, 640)], x_sh.at[pl.ds(s * 640, 640)])

    def _gcp(chunk, b):
        return pltpu.make_async_copy(x_sh.at[src_v.at[chunk]], xbufs[b], gsems[b])

    def _wcp(chunk, b):
        return pltpu.make_async_copy(
            w_hbm.at[pl.ds(base + chunk * CB, CB)], wbufs[b], wsems[b])

    def _scp_start(chunk, b):
        pltpu.async_copy(xbufs[b], acc.at[dst_v.at[chunk]], ssems[b], add=True)

    def _scp_wait(chunk, b):
        pltpu.make_async_copy(
            xbufs[b], acc.at[dst_v.at[chunk]], ssems[b]).wait()

    if do_scan:
        nbase = wid * ROWS_PER_TILE
        pltpu.sync_copy(x_hbm.at[pl.ds(nbase, ROWS_PER_TILE)], xv)
        pltpu.sync_copy(sel_hbm.at[pl.ds(nbase, ROWS_PER_TILE)], selv)
        pltpu.sync_copy(bat_hbm.at[pl.ds(nbase, ROWS_PER_TILE)], bv)

    def _z(i, _):
        zb[i, 0:16] = jnp.zeros((16,), jnp.float32)
        zb[i, 16:32] = jnp.zeros((16,), jnp.float32)
        return 0
    lax.fori_loop(0, CB, _z, 0)
    for q in range(ROWS_PER_TILE * 2 // CB):
        pltpu.sync_copy(zb, acc.at[pl.ds(s * 640 + q * CB, CB)])
    plsc.subcore_barrier()

    _gcp(0, 0).start()
    _wcp(0, 0).start()

    if do_scan:
        _scan_rows(wid, xv, selv, bv, pm)
        pltpu.sync_copy(pm, pmout_hbm.at[wid])

    def _mul(b):
        xb, wb = xbufs[b], wbufs[b]

        def _m(r, _):
            xb[r, 0:16] = xb[r, 0:16] * wb[r, 0:16]
            xb[r, 16:32] = xb[r, 16:32] * wb[r, 16:32]
            return 0
        lax.fori_loop(0, CB, _m, 0, unroll=8)

    def _outer(jj, _):
        for b in range(NBUF):
            chunk = jj * NBUF + b
            nxt = chunk + 1
            nb = (b + 1) % NBUF

            @pl.when(chunk >= NBUF - 1)
            def _():
                _scp_wait(chunk - (NBUF - 1), nb)

            @pl.when(nxt < NCHUNK)
            def _():
                _gcp(nxt, nb).start()
                _wcp(nxt, nb).start()
            _gcp(chunk, b).wait()
            _wcp(chunk, b).wait()
            _mul(b)
            _scp_start(chunk, b)
        return 0

    lax.fori_loop(0, NCHUNK // NBUF, _outer, 0)
    for tail in range(NCHUNK - (NBUF - 1), NCHUNK):
        _scp_wait(tail, tail % NBUF)

    plsc.subcore_barrier()
    pltpu.sync_copy(acc.at[pl.ds(s * 640, 640)],
                    out_hbm.at[c, pl.ds(s * 640, 640)])


def _conv_scratch():
    return ([
        pltpu.VMEM((NCHUNK, CB), jnp.int32),
        pltpu.VMEM((NCHUNK, CB), jnp.int32),
        pltpu.VMEM((CB, F), jnp.float32),
        pltpu.VMEM_SHARED((NPAD, F), jnp.float32),
        pltpu.VMEM_SHARED((NPAD, F), jnp.float32),
    ] + [pltpu.VMEM((CB, F), jnp.float32)] * (2 * NBUF)
      + [pltpu.SemaphoreType.DMA] * (3 * NBUF))


def _conv_sc(x_pad, w, src_t, dst_t):
    mesh = plsc.VectorSubcoreMesh(core_axis_name="c", subcore_axis_name="s")
    return pl.kernel(
        functools.partial(_conv_body, False),
        out_type=jax.ShapeDtypeStruct((2, NPAD, F), jnp.float32),
        mesh=mesh,
        compiler_params=pltpu.CompilerParams(use_tc_tiling_on_sc=False),
        scratch_types=_conv_scratch(),
    )(x_pad, w, src_t, dst_t)


def _conv_scan_sc(x_pad, sel1d, bat1d, w, src_t, dst_t):
    mesh = plsc.VectorSubcoreMesh(core_axis_name="c", subcore_axis_name="s")
    return pl.kernel(
        functools.partial(_conv_body, True),
        out_type=[
            jax.ShapeDtypeStruct((2, NPAD, F), jnp.float32),
            jax.ShapeDtypeStruct((NTILES, G * F), jnp.float32),
        ],
        mesh=mesh,
        compiler_params=pltpu.CompilerParams(use_tc_tiling_on_sc=False),
        scratch_types=[
            pltpu.VMEM((ROWS_PER_TILE, F), jnp.float32),
            pltpu.VMEM((ROWS_PER_TILE,), jnp.int32),
            pltpu.VMEM((ROWS_PER_TILE,), jnp.int32),
            pltpu.VMEM((G * F,), jnp.float32),
        ] + _conv_scratch(),
    )(x_pad, w, src_t, dst_t, sel1d, bat1d)


def _node_body(kk, part, batT, alive, sref, bref, pwref,
               xnext_ref, sel_ref, mean_ref):
    agg = part[0] + part[1]
    hb = jnp.maximum(agg, 0.0) * sref[...] + bref[...]
    pw = pwref[...]
    norm = jnp.sqrt(jnp.sum(pw * pw))
    sdot = lax.dot_general(pw.astype(jnp.bfloat16), hb.astype(jnp.bfloat16),
                           (((1,), (1,)), ((), ())),
                           preferred_element_type=jnp.float32)
    score = sdot / norm

    bits = lax.bitcast_convert_type(score, jnp.int32)
    key = jnp.where(bits < 0,
                    jnp.bitwise_xor(jnp.bitwise_not(bits), jnp.int32(INT_MIN)),
                    bits)
    key = jnp.where(alive[...] > 0, key, jnp.int32(INT_MIN))

    prefix = jnp.int32(INT_MIN)
    for b in range(31, -1, -1):
        if b == 31:
            cand = jnp.bitwise_xor(prefix, jnp.int32(INT_MIN))
        else:
            cand = jnp.bitwise_or(prefix, jnp.int32(1 << b))
        c = jnp.sum((key >= cand).astype(jnp.int32))
        prefix = jnp.where(c >= kk, cand, prefix)
    t = prefix

    gt = key > t
    eq = key == t
    need = jnp.int32(kk) - jnp.sum(gt.astype(jnp.int32))
    idx = lax.broadcasted_iota(jnp.int32, (1, NPAD), 1)
    pref = jnp.int32(0)
    for b in range(13, -1, -1):
        cand = jnp.bitwise_or(pref, jnp.int32(1 << b))
        c = jnp.sum((eq & (idx < cand)).astype(jnp.int32))
        pref = jnp.where(c < need, cand, pref)
    sel = gt | (eq & (idx <= pref) & (need > 0))

    mult = jnp.where(sel, jnp.tanh(score), 0.0)
    ones11 = jnp.ones((1, 1), jnp.float32)
    hi = jax.lax.Precision.HIGHEST
    multT = lax.dot_general(mult, ones11, (((0,), (0,)), ((), ())),
                            precision=hi,
                            preferred_element_type=jnp.float32)
    xnext = hb * multT
    xnext_ref[...] = xnext
    sel_ref[...] = sel.astype(jnp.int32)

    sel01 = sel.astype(jnp.float32)
    selT = lax.dot_general(sel01, ones11, (((0,), (0,)), ((), ())),
                           precision=hi,
                           preferred_element_type=jnp.float32)
    giota = lax.broadcasted_iota(jnp.int32, (NPAD, G), 1)
    onehot = (batT[...] == giota).astype(jnp.float32)
    sm = lax.dot_general(onehot, xnext, (((0,), (0,)), ((), ())),
                         precision=hi,
                         preferred_element_type=jnp.float32)
    cnt = lax.dot_general(onehot, selT, (((0,), (0,)), ((), ())),
                          precision=hi,
                          preferred_element_type=jnp.float32)
    mean_ref[...] = sm / jnp.maximum(cnt, 1.0)


def _node_tc(kk, part, batT, alive, s_i, b_i, pw_i):
    return pl.pallas_call(
        functools.partial(_node_body, kk),
        out_shape=[
            jax.ShapeDtypeStruct((NPAD, F), jnp.float32),
            jax.ShapeDtypeStruct((1, NPAD), jnp.int32),
            jax.ShapeDtypeStruct((G, F), jnp.float32),
        ],
    )(part, batT, alive, s_i, b_i, pw_i)


def _gmax_body(x_hbm, sel_hbm, bat_hbm, out_hbm, xv, selv, bv, pm):
    c = lax.axis_index("c")
    s = lax.axis_index("s")
    wid = s * 2 + c
    base = wid * ROWS_PER_TILE

    pltpu.sync_copy(x_hbm.at[pl.ds(base, ROWS_PER_TILE)], xv)
    pltpu.sync_copy(sel_hbm.at[pl.ds(base, ROWS_PER_TILE)], selv)
    pltpu.sync_copy(bat_hbm.at[pl.ds(base, ROWS_PER_TILE)], bv)

    _scan_rows(wid, xv, selv, bv, pm)

    pltpu.sync_copy(pm, out_hbm.at[wid])


def _gmax_sc(x_pad, sel1d, bat1d):
    mesh = plsc.VectorSubcoreMesh(core_axis_name="c", subcore_axis_name="s")
    return pl.kernel(
        _gmax_body,
        out_type=jax.ShapeDtypeStruct((NTILES, G * F), jnp.float32),
        mesh=mesh,
        compiler_params=pltpu.CompilerParams(use_tc_tiling_on_sc=False),
        scratch_types=[
            pltpu.VMEM((ROWS_PER_TILE, F), jnp.float32),
            pltpu.VMEM((ROWS_PER_TILE,), jnp.int32),
            pltpu.VMEM((ROWS_PER_TILE,), jnp.int32),
            pltpu.VMEM((G * F,), jnp.float32),
        ],
    )(x_pad, sel1d, bat1d)


def _head_body(pm0, pm1, pm2, pm3, mn0, mn1, mn2, mn3,
               w1, b1, s1, B1, w2, b2, s2, B2, w3, b3, o_ref):
    pieces = []
    for pm, mn in ((pm0, mn0), (pm1, mn1), (pm2, mn2), (pm3, mn3)):
        mx = jnp.max(pm[...], axis=0)
        mx = jnp.where(mx > jnp.float32(NEG_INF), mx, 0.0)
        pieces.append(mx)
        pieces.append(mn[...])
    z = jnp.concatenate(pieces, axis=1)
    h = jnp.maximum(_bdot(z, w1[...]) + b1[...], 0.0)
    h = h * s1[...] + B1[...]
    h = jnp.maximum(_bdot(h, w2[...]) + b2[...], 0.0)
    h = h * s2[...] + B2[...]
    o_ref[...] = _bdot(h, w3[...]) + b3[...]


def _head_tc(pms, mns, hw):
    return pl.pallas_call(
        _head_body,
        out_shape=jax.ShapeDtypeStruct((G, 1), jnp.float32),
    )(*pms, *mns, *hw)


def kernel(x, edge_index, batch, weight, params):
    p = params
    f32 = jnp.float32
    sqc = jnp.sqrt(jnp.float32(1.0 + 1e-5))

    ws, bs, ss, Bs = [], [], [], []
    for l in range(5):
        ws.append(p["gW%d" % l].T.astype(f32))
        bs.append(p["gb%d" % l].astype(f32).reshape(1, -1))
        if l < 4:
            ss.append((p["gg%d" % l] / sqc).reshape(1, -1))
            Bs.append(p["gB%d" % l].reshape(1, -1))

    e_pad = jnp.concatenate(
        [weight, jnp.zeros((EPAD - E, F), f32)], axis=0)
    w = _gmlp(e_pad, ws, bs, ss, Bs)

    src = edge_index[0].astype(jnp.int32)
    dst = edge_index[1].astype(jnp.int32)
    pad_idx = jnp.full((EPAD - E,), N, jnp.int32)
    src_t = jnp.concatenate([src, pad_idx]).reshape(NTILES, NCHUNK, CB)
    dst_t = jnp.concatenate([dst, pad_idx]).reshape(NTILES, NCHUNK, CB)

    x_pad = jnp.zeros((NPAD, F), f32).at[:N].set(x.astype(f32))
    bat1d = jnp.zeros((NPAD,), jnp.int32).at[:N].set(batch.astype(jnp.int32))
    batT = bat1d[:, None]
    alive = (jnp.arange(NPAD, dtype=jnp.int32)[None, :] < N).astype(jnp.int32)

    pms, mns = [], []
    xs = x_pad
    sel = None
    for i in range(4):
        if i == 0:
            part = _conv_sc(xs, w, src_t, dst_t)
        else:
            part, pm_prev = _conv_scan_sc(
                xs, sel.reshape(NPAD), bat1d, w, src_t, dst_t)
            pms.append(pm_prev.reshape(NTILES, G, F))
        s_i = (p["bng%d" % (i + 1)] / sqc).reshape(1, F)
        b_i = p["bnb%d" % (i + 1)].reshape(1, F)
        pw_i = p["pw%d" % (i + 1)].reshape(1, F).astype(f32)
        xs, sel, mean_i = _node_tc(KS[i], part, batT, alive, s_i, b_i, pw_i)
        alive = sel
        mns.append(mean_i)
    pms.append(_gmax_sc(xs, sel.reshape(NPAD), bat1d).reshape(NTILES, G, F))

    hw = [
        p["lin1W"].T.astype(f32), p["lin1b"].reshape(1, -1),
        (p["bn1g"] / sqc).reshape(1, -1), p["bn1b"].reshape(1, -1),
        p["lin2W"].T.astype(f32), p["lin2b"].reshape(1, -1),
        (p["bn2g"] / sqc).reshape(1, -1), p["bn2b"].reshape(1, -1),
        p["lin3W"].T.astype(f32), p["lin3b"].reshape(1, -1),
    ]
    out = _head_tc(pms, mns, hw)
    return out.reshape(-1)


# speedup vs baseline: 1.0940x; 1.0000x over previous
import functools

import jax
import jax.numpy as jnp
from jax import lax
from jax.experimental import pallas as pl
from jax.experimental.pallas import tpu as pltpu
from jax.experimental.pallas import tpu_sc as plsc

N = 10000
NPAD = 10240
E = 160000
EPAD = 163840
G = 128
F = 32
NTILES = 32
EDGES_PER_TILE = EPAD // NTILES
NCHUNK = 40
CB = 128
ROWS_PER_TILE = NPAD // NTILES
KS = [5000, 4000, 3200, 2560]
NEG_INF = float("-inf")
INT_MIN = -2147483648


def _bdot(a, b):
    return jnp.dot(a.astype(jnp.bfloat16), b.astype(jnp.bfloat16),
                   preferred_element_type=jnp.float32)


def _gmlp_body(e_ref, w0, w1, w2, w3, w4, b0, b1, b2, b3, b4,
               s0, s1, s2, s3, B0, B1, B2, B3, o_ref):
    h = e_ref[...]
    wsr = (w0, w1, w2, w3)
    bsr = (b0, b1, b2, b3)
    ssr = (s0, s1, s2, s3)
    Bsr = (B0, B1, B2, B3)
    for l in range(4):
        h = _bdot(h, wsr[l][...]) + bsr[l][...]
        h = h * ssr[l][...] + Bsr[l][...]
        h = jnp.maximum(h, 0.0)
    o_ref[...] = _bdot(h, w4[...]) + b4[...]


def _gmlp(e_pad, ws, bs, ss, Bs):
    blk = 1024
    grid = EPAD // blk
    full = lambda shape: pl.BlockSpec(shape, lambda i: (0, 0))
    return pl.pallas_call(
        _gmlp_body,
        grid=(grid,),
        in_specs=[pl.BlockSpec((blk, F), lambda i: (i, 0))]
        + [full(w.shape) for w in ws] + [full(b.shape) for b in bs]
        + [full(s.shape) for s in ss] + [full(B.shape) for B in Bs],
        out_specs=pl.BlockSpec((blk, F), lambda i: (i, 0)),
        out_shape=jax.ShapeDtypeStruct((EPAD, F), jnp.float32),
        compiler_params=pltpu.CompilerParams(
            dimension_semantics=("arbitrary",)),
    )(e_pad, *ws, *bs, *ss, *Bs)


NBUF = 4


def _scan_rows(wid, xv, selv, bv, pm):
    ninf = jnp.full((16,), NEG_INF, jnp.float32)

    def _init(i, _):
        pm[pl.ds(i * 16, 16)] = ninf
        return 0
    lax.fori_loop(0, G * F // 16, _init, 0)

    def _group(gi, carry):
        cur_g, m0, m1 = carry
        vb = bv[pl.ds(gi * 16, 16)]
        vs = selv[pl.ds(gi * 16, 16)]
        for j in range(16):
            r = gi * 16 + j
            g = vb[j]
            svaln = vs[j]
            x0 = xv[r, 0:16]
            x1 = xv[r, 16:32]
            x0 = jnp.where(svaln > 0, x0, ninf)
            x1 = jnp.where(svaln > 0, x1, ninf)
            is_new = g != cur_g

            @pl.when(is_new & (cur_g >= 0))
            def _():
                pm[pl.ds(cur_g * F, 16)] = m0
                pm[pl.ds(cur_g * F + 16, 16)] = m1

            m0 = jnp.where(is_new, x0, jnp.maximum(m0, x0))
            m1 = jnp.where(is_new, x1, jnp.maximum(m1, x1))
            cur_g = g
        return (cur_g, m0, m1)

    cur_g, m0, m1 = lax.fori_loop(
        0, ROWS_PER_TILE // 16, _group, (jnp.int32(-1), ninf, ninf))

    @pl.when(cur_g >= 0)
    def _():
        pm[pl.ds(cur_g * F, 16)] = m0
        pm[pl.ds(cur_g * F + 16, 16)] = m1


def _conv_body(do_scan, *refs):
    if do_scan:
        (x_hbm, w_hbm, src_hbm, dst_hbm, sel_hbm, bat_hbm,
         out_hbm, pmout_hbm) = refs[:8]
        refs = refs[8:]
        xv, selv, bv, pm = refs[:4]
        refs = refs[4:]
    else:
        x_hbm, w_hbm, src_hbm, dst_hbm, out_hbm = refs[:5]
        refs = refs[5:]
    src_v, dst_v, zb, acc, x_sh = refs[:5]
    refs = refs[5:]
    xbufs = refs[0:NBUF]
    wbufs = refs[NBUF:2 * NBUF]
    gsems = refs[2 * NBUF:3 * NBUF]
    wsems = refs[3 * NBUF:4 * NBUF]
    ssems = refs[4 * NBUF:5 * NBUF]

    c = lax.axis_index("c")
    s = lax.axis_index("s")
    wid = s * 2 + c
    base = wid * EDGES_PER_TILE

    pltpu.sync_copy(src_hbm.at[wid], src_v)
    pltpu.sync_copy(dst_hbm.at[wid], dst_v)
    pltpu.sync_copy(x_hbm.at[pl.ds(s * 640, 640)], x_sh.at[pl.ds(s * 640, 640)])

    def _gcp(chunk, b):
        return pltpu.make_async_copy(x_sh.at[src_v.at[chunk]], xbufs[b], gsems[b])

    def _wcp(chunk, b):
        return pltpu.make_async_copy(
            w_hbm.at[pl.ds(base + chunk * CB, CB)], wbufs[b], wsems[b])

    def _scp_start(chunk, b):
        pltpu.async_copy(xbufs[b], acc.at[dst_v.at[chunk]], ssems[b], add=True)

    def _scp_wait(chunk, b):
        pltpu.make_async_copy(
            xbufs[b], acc.at[dst_v.at[chunk]], ssems[b]).wait()

    if do_scan:
        nbase = wid * ROWS_PER_TILE
        pltpu.sync_copy(x_hbm.at[pl.ds(nbase, ROWS_PER_TILE)], xv)
        pltpu.sync_copy(sel_hbm.at[pl.ds(nbase, ROWS_PER_TILE)], selv)
        pltpu.sync_copy(bat_hbm.at[pl.ds(nbase, ROWS_PER_TILE)], bv)

    def _z(i, _):
        zb[i, 0:16] = jnp.zeros((16,), jnp.float32)
        zb[i, 16:32] = jnp.zeros((16,), jnp.float32)
        return 0
    lax.fori_loop(0, CB, _z, 0)
    for q in range(ROWS_PER_TILE * 2 // CB):
        pltpu.sync_copy(zb, acc.at[pl.ds(s * 640 + q * CB, CB)])
    plsc.subcore_barrier()

    _gcp(0, 0).start()
    _wcp(0, 0).start()

    if do_scan:
        _scan_rows(wid, xv, selv, bv, pm)
        pltpu.sync_copy(pm, pmout_hbm.at[wid])

    def _mul(b):
        xb, wb = xbufs[b], wbufs[b]

        def _m(r, _):
            xb[r, 0:16] = xb[r, 0:16] * wb[r, 0:16]
            xb[r, 16:32] = xb[r, 16:32] * wb[r, 16:32]
            return 0
        lax.fori_loop(0, CB, _m, 0, unroll=8)

    def _outer(jj, _):
        for b in range(NBUF):
            chunk = jj * NBUF + b
            nxt = chunk + 1
            nb = (b + 1) % NBUF

            @pl.when(chunk >= NBUF - 1)
            def _():
                _scp_wait(chunk - (NBUF - 1), nb)

            @pl.when(nxt < NCHUNK)
            def _():
                _gcp(nxt, nb).start()
                _wcp(nxt, nb).start()
            _gcp(chunk, b).wait()
            _wcp(chunk, b).wait()
            _mul(b)
            _scp_start(chunk, b)
        return 0

    lax.fori_loop(0, NCHUNK // NBUF, _outer, 0)
    for tail in range(NCHUNK - (NBUF - 1), NCHUNK):
        _scp_wait(tail, tail % NBUF)

    plsc.subcore_barrier()
    pltpu.sync_copy(acc.at[pl.ds(s * 640, 640)],
                    out_hbm.at[c, pl.ds(s * 640, 640)])


def _conv_scratch():
    return ([
        pltpu.VMEM((NCHUNK, CB), jnp.int32),
        pltpu.VMEM((NCHUNK, CB), jnp.int32),
        pltpu.VMEM((CB, F), jnp.float32),
        pltpu.VMEM_SHARED((NPAD, F), jnp.float32),
        pltpu.VMEM_SHARED((NPAD, F), jnp.float32),
    ] + [pltpu.VMEM((CB, F), jnp.float32)] * (2 * NBUF)
      + [pltpu.SemaphoreType.DMA] * (3 * NBUF))


def _conv_sc(x_pad, w, src_t, dst_t):
    mesh = plsc.VectorSubcoreMesh(core_axis_name="c", subcore_axis_name="s")
    return pl.kernel(
        functools.partial(_conv_body, False),
        out_type=jax.ShapeDtypeStruct((2, NPAD, F), jnp.float32),
        mesh=mesh,
        compiler_params=pltpu.CompilerParams(use_tc_tiling_on_sc=False),
        scratch_types=_conv_scratch(),
    )(x_pad, w, src_t, dst_t)


def _conv_scan_sc(x_pad, sel1d, bat1d, w, src_t, dst_t):
    mesh = plsc.VectorSubcoreMesh(core_axis_name="c", subcore_axis_name="s")
    return pl.kernel(
        functools.partial(_conv_body, True),
        out_type=[
            jax.ShapeDtypeStruct((2, NPAD, F), jnp.float32),
            jax.ShapeDtypeStruct((NTILES, G * F), jnp.float32),
        ],
        mesh=mesh,
        compiler_params=pltpu.CompilerParams(use_tc_tiling_on_sc=False),
        scratch_types=[
            pltpu.VMEM((ROWS_PER_TILE, F), jnp.float32),
            pltpu.VMEM((ROWS_PER_TILE,), jnp.int32),
            pltpu.VMEM((ROWS_PER_TILE,), jnp.int32),
            pltpu.VMEM((G * F,), jnp.float32),
        ] + _conv_scratch(),
    )(x_pad, w, src_t, dst_t, sel1d, bat1d)


def _node_body(kk, part, batT, alive, sref, bref, pwref,
               xnext_ref, sel_ref, mean_ref):
    agg = part[0] + part[1]
    hb = jnp.maximum(agg, 0.0) * sref[...] + bref[...]
    pw = pwref[...]
    norm = jnp.sqrt(jnp.sum(pw * pw))
    sdot = lax.dot_general(pw.astype(jnp.bfloat16), hb.astype(jnp.bfloat16),
                           (((1,), (1,)), ((), ())),
                           preferred_element_type=jnp.float32)
    score = sdot / norm

    bits = lax.bitcast_convert_type(score, jnp.int32)
    key = jnp.where(bits < 0,
                    jnp.bitwise_xor(jnp.bitwise_not(bits), jnp.int32(INT_MIN)),
                    bits)
    key = jnp.where(alive[...] > 0, key, jnp.int32(INT_MIN))

    prefix = jnp.int32(INT_MIN)
    for b in range(31, -1, -1):
        if b == 31:
            cand = jnp.bitwise_xor(prefix, jnp.int32(INT_MIN))
        else:
            cand = jnp.bitwise_or(prefix, jnp.int32(1 << b))
        c = jnp.sum((key >= cand).astype(jnp.int32))
        prefix = jnp.where(c >= kk, cand, prefix)
    t = prefix

    gt = key > t
    eq = key == t
    need = jnp.int32(kk) - jnp.sum(gt.astype(jnp.int32))
    idx = lax.broadcasted_iota(jnp.int32, (1, NPAD), 1)
    pref = jnp.int32(0)
    for b in range(13, -1, -1):
        cand = jnp.bitwise_or(pref, jnp.int32(1 << b))
        c = jnp.sum((eq & (idx < cand)).astype(jnp.int32))
        pref = jnp.where(c < need, cand, pref)
    sel = gt | (eq & (idx <= pref) & (need > 0))

    mult = jnp.where(sel, jnp.tanh(score), 0.0)
    ones11 = jnp.ones((1, 1), jnp.float32)
    hi = jax.lax.Precision.HIGHEST
    multT = lax.dot_general(mult, ones11, (((0,), (0,)), ((), ())),
                            precision=hi,
                            preferred_element_type=jnp.float32)
    xnext = hb * multT
    xnext_ref[...] = xnext
    sel_ref[...] = sel.astype(jnp.int32)

    sel01 = sel.astype(jnp.float32)
    selT = lax.dot_general(sel01, ones11, (((0,), (0,)), ((), ())),
                           precision=hi,
                           preferred_element_type=jnp.float32)
    giota = lax.broadcasted_iota(jnp.int32, (NPAD, G), 1)
    onehot = (batT[...] == giota).astype(jnp.float32)
    sm = lax.dot_general(onehot, xnext, (((0,), (0,)), ((), ())),
                         precision=hi,
                         preferred_element_type=jnp.float32)
    cnt = lax.dot_general(onehot, selT, (((0,), (0,)), ((), ())),
                          precision=hi,
                          preferred_element_type=jnp.float32)
    mean_ref[...] = sm / jnp.maximum(cnt, 1.0)


def _node_tc(kk, part, batT, alive, s_i, b_i, pw_i):
    return pl.pallas_call(
        functools.partial(_node_body, kk),
        out_shape=[
            jax.ShapeDtypeStruct((NPAD, F), jnp.float32),
            jax.ShapeDtypeStruct((1, NPAD), jnp.int32),
            jax.ShapeDtypeStruct((G, F), jnp.float32),
        ],
    )(part, batT, alive, s_i, b_i, pw_i)


def _gmax_body(x_hbm, sel_hbm, bat_hbm, out_hbm, xv, selv, bv, pm):
    c = lax.axis_index("c")
    s = lax.axis_index("s")
    wid = s * 2 + c
    base = wid * ROWS_PER_TILE

    pltpu.sync_copy(x_hbm.at[pl.ds(base, ROWS_PER_TILE)], xv)
    pltpu.sync_copy(sel_hbm.at[pl.ds(base, ROWS_PER_TILE)], selv)
    pltpu.sync_copy(bat_hbm.at[pl.ds(base, ROWS_PER_TILE)], bv)

    _scan_rows(wid, xv, selv, bv, pm)

    pltpu.sync_copy(pm, out_hbm.at[wid])


def _gmax_sc(x_pad, sel1d, bat1d):
    mesh = plsc.VectorSubcoreMesh(core_axis_name="c", subcore_axis_name="s")
    return pl.kernel(
        _gmax_body,
        out_type=jax.ShapeDtypeStruct((NTILES, G * F), jnp.float32),
        mesh=mesh,
        compiler_params=pltpu.CompilerParams(use_tc_tiling_on_sc=False),
        scratch_types=[
            pltpu.VMEM((ROWS_PER_TILE, F), jnp.float32),
            pltpu.VMEM((ROWS_PER_TILE,), jnp.int32),
            pltpu.VMEM((ROWS_PER_TILE,), jnp.int32),
            pltpu.VMEM((G * F,), jnp.float32),
        ],
    )(x_pad, sel1d, bat1d)


def _head_body(pm0, pm1, pm2, pm3, mn0, mn1, mn2, mn3,
               w1, b1, s1, B1, w2, b2, s2, B2, w3, b3, o_ref):
    pieces = []
    for pm, mn in ((pm0, mn0), (pm1, mn1), (pm2, mn2), (pm3, mn3)):
        mx = jnp.max(pm[...], axis=0)
        mx = jnp.where(mx > jnp.float32(NEG_INF), mx, 0.0)
        pieces.append(mx)
        pieces.append(mn[...])
    z = jnp.concatenate(pieces, axis=1)
    h = jnp.maximum(_bdot(z, w1[...]) + b1[...], 0.0)
    h = h * s1[...] + B1[...]
    h = jnp.maximum(_bdot(h, w2[...]) + b2[...], 0.0)
    h = h * s2[...] + B2[...]
    o_ref[...] = _bdot(h, w3[...]) + b3[...]


def _head_tc(pms, mns, hw):
    return pl.pallas_call(
        _head_body,
        out_shape=jax.ShapeDtypeStruct((G, 1), jnp.float32),
    )(*pms, *mns, *hw)


def kernel(x, edge_index, batch, weight, params):
    p = params
    f32 = jnp.float32
    sqc = jnp.sqrt(jnp.float32(1.0 + 1e-5))

    ws, bs, ss, Bs = [], [], [], []
    for l in range(5):
        ws.append(p["gW%d" % l].T.astype(f32))
        bs.append(p["gb%d" % l].astype(f32).reshape(1, -1))
        if l < 4:
            ss.append((p["gg%d" % l] / sqc).reshape(1, -1))
            Bs.append(p["gB%d" % l].reshape(1, -1))

    e_pad = jnp.concatenate(
        [weight, jnp.zeros((EPAD - E, F), f32)], axis=0)
    w = _gmlp(e_pad, ws, bs, ss, Bs)

    src = edge_index[0].astype(jnp.int32)
    dst = edge_index[1].astype(jnp.int32)
    pad_idx = jnp.full((EPAD - E,), N, jnp.int32)
    src_t = jnp.concatenate([src, pad_idx]).reshape(NTILES, NCHUNK, CB)
    dst_t = jnp.concatenate([dst, pad_idx]).reshape(NTILES, NCHUNK, CB)

    x_pad = jnp.zeros((NPAD, F), f32).at[:N].set(x.astype(f32))
    bat1d = jnp.zeros((NPAD,), jnp.int32).at[:N].set(batch.astype(jnp.int32))
    batT = bat1d[:, None]
    alive = (jnp.arange(NPAD, dtype=jnp.int32)[None, :] < N).astype(jnp.int32)

    pms, mns = [], []
    xs = x_pad
    sel = None
    for i in range(4):
        if i == 0:
            part = _conv_sc(xs, w, src_t, dst_t)
        else:
            part, pm_prev = _conv_scan_sc(
                xs, sel.reshape(NPAD), bat1d, w, src_t, dst_t)
            pms.append(pm_prev.reshape(NTILES, G, F))
        s_i = (p["bng%d" % (i + 1)] / sqc).reshape(1, F)
        b_i = p["bnb%d" % (i + 1)].reshape(1, F)
        pw_i = p["pw%d" % (i + 1)].reshape(1, F).astype(f32)
        xs, sel, mean_i = _node_tc(KS[i], part, batT, alive, s_i, b_i, pw_i)
        alive = sel
        mns.append(mean_i)
    pms.append(_gmax_sc(xs, sel.reshape(NPAD), bat1d).reshape(NTILES, G, F))

    hw = [
        p["lin1W"].T.astype(f32), p["lin1b"].reshape(1, -1),
        (p["bn1g"] / sqc).reshape(1, -1), p["bn1b"].reshape(1, -1),
        p["lin2W"].T.astype(f32), p["lin2b"].reshape(1, -1),
        (p["bn2g"] / sqc).reshape(1, -1), p["bn2b"].reshape(1, -1),
        p["lin3W"].T.astype(f32), p["lin3b"].reshape(1, -1),
    ]
    out = _head_tc(pms, mns, hw)
    return out.reshape(-1)
